# initial kernel scaffold (unmeasured)
import jax
import jax.numpy as jnp
from jax import lax
from jax.experimental import pallas as pl
from jax.experimental.pallas import tpu as pltpu

M = 8192
D = 2048
NCHUNK = 16
CM = M // NCHUNK


def kernel(partial, resid, gamma):
    gamma2d = gamma.reshape(1, D)

    def body(partial_ref, resid_ref, gamma_ref, out_ref,
             peer_ref, a_ref, b_ref, r_ref, o_ref, g_ref,
             send_sem, recv_sem, cp_sems):
        x = lax.axis_index("x")
        y = lax.axis_index("y")
        z = lax.axis_index("z")
        partner = (1 - x, y, z)

        bar = pltpu.get_barrier_semaphore()
        pl.semaphore_signal(bar, inc=1, device_id=partner,
                            device_id_type=pl.DeviceIdType.MESH)
        pl.semaphore_wait(bar, 1)

        rdma = pltpu.make_async_remote_copy(
            src_ref=partial_ref.at[0],
            dst_ref=peer_ref,
            send_sem=send_sem,
            recv_sem=recv_sem,
            device_id=partner,
            device_id_type=pl.DeviceIdType.MESH,
        )
        rdma.start()

        gcp = pltpu.make_async_copy(gamma_ref, g_ref, cp_sems.at[3])
        gcp.start()
        gcp.wait()

        rdma.wait()

        for k in range(NCHUNK):
            rows = pl.ds(k * CM, CM)
            c_a = pltpu.make_async_copy(partial_ref.at[0, rows], a_ref,
                                        cp_sems.at[0])
            c_b = pltpu.make_async_copy(peer_ref.at[rows], b_ref,
                                        cp_sems.at[1])
            c_r = pltpu.make_async_copy(resid_ref.at[rows], r_ref,
                                        cp_sems.at[2])
            c_a.start()
            c_b.start()
            c_r.start()
            c_a.wait()
            c_b.wait()
            c_r.wait()
            yv = a_ref[...] + b_ref[...] + r_ref[...]
            rms = jnp.sqrt(jnp.mean(yv * yv, axis=-1, keepdims=True) + 1e-6)
            o_ref[...] = yv / rms * g_ref[...]
            c_o = pltpu.make_async_copy(o_ref, out_ref.at[rows], cp_sems.at[3])
            c_o.start()
            c_o.wait()

    return pl.pallas_call(
        body,
        out_shape=jax.ShapeDtypeStruct((M, D), jnp.float32),
        in_specs=[
            pl.BlockSpec(memory_space=pltpu.ANY),
            pl.BlockSpec(memory_space=pltpu.ANY),
            pl.BlockSpec(memory_space=pltpu.ANY),
        ],
        out_specs=pl.BlockSpec(memory_space=pltpu.ANY),
        scratch_shapes=[
            pltpu.ANY((M, D), jnp.float32),
            pltpu.VMEM((CM, D), jnp.float32),
            pltpu.VMEM((CM, D), jnp.float32),
            pltpu.VMEM((CM, D), jnp.float32),
            pltpu.VMEM((CM, D), jnp.float32),
            pltpu.VMEM((1, D), jnp.float32),
            pltpu.SemaphoreType.DMA,
            pltpu.SemaphoreType.DMA,
            pltpu.SemaphoreType.DMA((4,)),
        ],
        compiler_params=pltpu.CompilerParams(collective_id=0),
    )(partial, resid, gamma2d)


# baseline (device time: 893199 ns/iter reference)
import jax
import jax.numpy as jnp
from jax import lax
from jax.experimental import pallas as pl
from jax.experimental.pallas import tpu as pltpu

M = 8192
D = 2048
NCHUNK = 16
CM = M // NCHUNK


def kernel(partial, resid, gamma):
    gamma2d = gamma.reshape(1, D)

    def body(partial_ref, resid_ref, gamma_ref, out_ref,
             peer_ref, a_ref, b_ref, r_ref, o_ref, g_ref,
             send_sem, recv_sem, cp_sems):
        x = lax.axis_index("x")
        y = lax.axis_index("y")
        z = lax.axis_index("z")
        partner = (1 - x, y, z)

        bar = pltpu.get_barrier_semaphore()
        pl.semaphore_signal(bar, inc=1, device_id=partner,
                            device_id_type=pl.DeviceIdType.MESH)
        pl.semaphore_wait(bar, 1)

        rdma = pltpu.make_async_remote_copy(
            src_ref=partial_ref.at[0],
            dst_ref=peer_ref,
            send_sem=send_sem,
            recv_sem=recv_sem,
            device_id=partner,
            device_id_type=pl.DeviceIdType.MESH,
        )
        rdma.start()

        gcp = pltpu.make_async_copy(gamma_ref, g_ref, cp_sems.at[3])
        gcp.start()
        gcp.wait()

        rdma.wait()

        for k in range(NCHUNK):
            rows = pl.ds(k * CM, CM)
            c_a = pltpu.make_async_copy(partial_ref.at[0, rows], a_ref,
                                        cp_sems.at[0])
            c_b = pltpu.make_async_copy(peer_ref.at[rows], b_ref,
                                        cp_sems.at[1])
            c_r = pltpu.make_async_copy(resid_ref.at[rows], r_ref,
                                        cp_sems.at[2])
            c_a.start()
            c_b.start()
            c_r.start()
            c_a.wait()
            c_b.wait()
            c_r.wait()
            yv = a_ref[...] + b_ref[...] + r_ref[...]
            rms = jnp.sqrt(jnp.mean(yv * yv, axis=-1, keepdims=True) + 1e-6)
            o_ref[...] = yv / rms * g_ref[...]
            c_o = pltpu.make_async_copy(o_ref, out_ref.at[rows], cp_sems.at[3])
            c_o.start()
            c_o.wait()

    out, _ = pl.pallas_call(
        body,
        out_shape=(
            jax.ShapeDtypeStruct((M, D), jnp.float32),
            jax.ShapeDtypeStruct((M, D), jnp.float32),
        ),
        in_specs=[
            pl.BlockSpec(memory_space=pl.ANY),
            pl.BlockSpec(memory_space=pl.ANY),
            pl.BlockSpec(memory_space=pl.ANY),
        ],
        out_specs=(
            pl.BlockSpec(memory_space=pl.ANY),
            pl.BlockSpec(memory_space=pl.ANY),
        ),
        scratch_shapes=[
            pltpu.VMEM((CM, D), jnp.float32),
            pltpu.VMEM((CM, D), jnp.float32),
            pltpu.VMEM((CM, D), jnp.float32),
            pltpu.VMEM((CM, D), jnp.float32),
            pltpu.VMEM((1, D), jnp.float32),
            pltpu.SemaphoreType.DMA,
            pltpu.SemaphoreType.DMA,
            pltpu.SemaphoreType.DMA((4,)),
        ],
        compiler_params=pltpu.CompilerParams(collective_id=0),
    )(partial, resid, gamma2d)
    return out


# device time: 477599 ns/iter; 1.8702x vs baseline; 1.8702x over previous
import jax
import jax.numpy as jnp
from jax import lax
from jax.experimental import pallas as pl
from jax.experimental.pallas import tpu as pltpu

M = 8192
D = 2048
NRING = 16
CM = M // NRING
N_CW = NRING // 2
N_CCW = NRING // 2 - 1

CYC = [(0, 0), (0, 1), (0, 2), (0, 3),
       (1, 3), (1, 2), (1, 1), (2, 1),
       (2, 2), (2, 3), (3, 3), (3, 2),
       (3, 1), (3, 0), (2, 0), (1, 0)]


def kernel(partial, resid, gamma):
    gamma2d = gamma.reshape(1, D)

    def body(partial_ref, resid_ref, gamma_ref, out_ref,
             a_ref, b_ref, r_ref, o_ref, g_ref,
             x_send, x_recv, cp_sems,
             cw_send, cw_recv, ccw_send, ccw_recv):
        x = lax.axis_index("x")
        yy = lax.axis_index("y")
        zz = lax.axis_index("z")

        R = jnp.int32(0)
        ry = jnp.int32(0)
        rz = jnp.int32(0)
        ly = jnp.int32(0)
        lz = jnp.int32(0)
        for k, (cy, cz) in enumerate(CYC):
            m = ((yy == cy) & (zz == cz)).astype(jnp.int32)
            nxt_y, nxt_z = CYC[(k + 1) % NRING]
            prv_y, prv_z = CYC[(k - 1) % NRING]
            R = R + k * m
            ry = ry + nxt_y * m
            rz = rz + nxt_z * m
            ly = ly + prv_y * m
            lz = lz + prv_z * m

        partner = (1 - x, yy, zz)
        right = (x, ry, rz)
        left = (x, ly, lz)

        bar = pltpu.get_barrier_semaphore()
        for nbr in (partner, right, left):
            pl.semaphore_signal(bar, inc=1, device_id=nbr,
                                device_id_type=pl.DeviceIdType.MESH)
        pl.semaphore_wait(bar, 3)

        myrows = pl.ds(R * CM, CM)

        xrd = pltpu.make_async_remote_copy(
            src_ref=partial_ref.at[0, myrows],
            dst_ref=b_ref,
            send_sem=x_send,
            recv_sem=x_recv,
            device_id=partner,
            device_id_type=pl.DeviceIdType.MESH,
        )
        xrd.start()

        c_a = pltpu.make_async_copy(partial_ref.at[0, myrows], a_ref,
                                    cp_sems.at[0])
        c_r = pltpu.make_async_copy(resid_ref.at[myrows], r_ref,
                                    cp_sems.at[1])
        c_g = pltpu.make_async_copy(gamma_ref, g_ref, cp_sems.at[2])
        c_a.start()
        c_r.start()
        c_g.start()
        c_a.wait()
        c_r.wait()
        c_g.wait()
        xrd.wait()

        yv = a_ref[...] + b_ref[...] + r_ref[...]
        rms = jnp.sqrt(jnp.mean(yv * yv, axis=-1, keepdims=True) + 1e-6)
        o_ref[...] = yv / rms * g_ref[...]
        c_o = pltpu.make_async_copy(o_ref, out_ref.at[myrows], cp_sems.at[3])
        c_o.start()
        c_o.wait()

        def chunk_rows(c):
            return pl.ds(c * CM, CM)

        sends = []
        for h in range(N_CW):
            c_cw = jnp.mod(R - h, NRING)
            s = pltpu.make_async_remote_copy(
                src_ref=out_ref.at[chunk_rows(c_cw)],
                dst_ref=out_ref.at[chunk_rows(c_cw)],
                send_sem=cw_send.at[h],
                recv_sem=cw_recv.at[h],
                device_id=right,
                device_id_type=pl.DeviceIdType.MESH,
            )
            s.start()
            sends.append(s)
            if h < N_CCW:
                c_ccw = jnp.mod(R + h, NRING)
                s2 = pltpu.make_async_remote_copy(
                    src_ref=out_ref.at[chunk_rows(c_ccw)],
                    dst_ref=out_ref.at[chunk_rows(c_ccw)],
                    send_sem=ccw_send.at[h],
                    recv_sem=ccw_recv.at[h],
                    device_id=left,
                    device_id_type=pl.DeviceIdType.MESH,
                )
                s2.start()
                sends.append(s2)

            rc = jnp.mod(R - 1 - h, NRING)
            w = pltpu.make_async_remote_copy(
                src_ref=out_ref.at[chunk_rows(rc)],
                dst_ref=out_ref.at[chunk_rows(rc)],
                send_sem=cw_send.at[h],
                recv_sem=cw_recv.at[h],
                device_id=left,
                device_id_type=pl.DeviceIdType.MESH,
            )
            w.wait_recv()
            if h < N_CCW:
                rc2 = jnp.mod(R + 1 + h, NRING)
                w2 = pltpu.make_async_remote_copy(
                    src_ref=out_ref.at[chunk_rows(rc2)],
                    dst_ref=out_ref.at[chunk_rows(rc2)],
                    send_sem=ccw_send.at[h],
                    recv_sem=ccw_recv.at[h],
                    device_id=right,
                    device_id_type=pl.DeviceIdType.MESH,
                )
                w2.wait_recv()

        for s in sends:
            s.wait_send()

    return pl.pallas_call(
        body,
        out_shape=jax.ShapeDtypeStruct((M, D), jnp.float32),
        in_specs=[
            pl.BlockSpec(memory_space=pl.ANY),
            pl.BlockSpec(memory_space=pl.ANY),
            pl.BlockSpec(memory_space=pl.ANY),
        ],
        out_specs=pl.BlockSpec(memory_space=pl.ANY),
        scratch_shapes=[
            pltpu.VMEM((CM, D), jnp.float32),
            pltpu.VMEM((CM, D), jnp.float32),
            pltpu.VMEM((CM, D), jnp.float32),
            pltpu.VMEM((CM, D), jnp.float32),
            pltpu.VMEM((1, D), jnp.float32),
            pltpu.SemaphoreType.DMA,
            pltpu.SemaphoreType.DMA,
            pltpu.SemaphoreType.DMA((4,)),
            pltpu.SemaphoreType.DMA((N_CW,)),
            pltpu.SemaphoreType.DMA((N_CW,)),
            pltpu.SemaphoreType.DMA((N_CCW,)),
            pltpu.SemaphoreType.DMA((N_CCW,)),
        ],
        compiler_params=pltpu.CompilerParams(collective_id=0),
    )(partial, resid, gamma2d)


# device time: 387020 ns/iter; 2.3079x vs baseline; 1.2340x over previous
import jax
import jax.numpy as jnp
from jax import lax
from jax.experimental import pallas as pl
from jax.experimental.pallas import tpu as pltpu

M = 8192
D = 2048
NRING = 16
CM = M // NRING
N_CW = NRING // 2
N_CCW = NRING // 2 - 1

XEX = 152
SH = CM - 2 * XEX
RNG = XEX + SH

CYC = [(0, 0), (0, 1), (0, 2), (0, 3),
       (1, 3), (1, 2), (1, 1), (2, 1),
       (2, 2), (2, 3), (3, 3), (3, 2),
       (3, 1), (3, 0), (2, 0), (1, 0)]


def kernel(partial, resid, gamma):
    gamma2d = gamma.reshape(1, D)

    def body(partial_ref, resid_ref, gamma_ref, out_ref,
             a_ref, b_ref, r_ref, o_ref, g_ref,
             x_send, x_recv, cp_sems,
             cw_send, cw_recv, ccw_send, ccw_recv,
             xp_cw_send, xp_cw_recv, xp_ccw_send, xp_ccw_recv):
        x = lax.axis_index("x")
        yy = lax.axis_index("y")
        zz = lax.axis_index("z")

        R = jnp.int32(0)
        ry = jnp.int32(0)
        rz = jnp.int32(0)
        ly = jnp.int32(0)
        lz = jnp.int32(0)
        for k, (cy, cz) in enumerate(CYC):
            m = ((yy == cy) & (zz == cz)).astype(jnp.int32)
            nxt_y, nxt_z = CYC[(k + 1) % NRING]
            prv_y, prv_z = CYC[(k - 1) % NRING]
            R = R + k * m
            ry = ry + nxt_y * m
            rz = rz + nxt_z * m
            ly = ly + prv_y * m
            lz = lz + prv_z * m

        partner = (1 - x, yy, zz)
        right = (x, ry, rz)
        left = (x, ly, lz)

        bar = pltpu.get_barrier_semaphore()
        for nbr in (partner, right, left):
            pl.semaphore_signal(bar, inc=1, device_id=nbr,
                                device_id_type=pl.DeviceIdType.MESH)
        pl.semaphore_wait(bar, 3)

        myrows = pl.ds(R * CM, CM)

        xrd = pltpu.make_async_remote_copy(
            src_ref=partial_ref.at[0, myrows],
            dst_ref=b_ref,
            send_sem=x_send,
            recv_sem=x_recv,
            device_id=partner,
            device_id_type=pl.DeviceIdType.MESH,
        )
        xrd.start()

        c_a = pltpu.make_async_copy(partial_ref.at[0, myrows], a_ref,
                                    cp_sems.at[0])
        c_r = pltpu.make_async_copy(resid_ref.at[myrows], r_ref,
                                    cp_sems.at[1])
        c_g = pltpu.make_async_copy(gamma_ref, g_ref, cp_sems.at[2])
        c_a.start()
        c_r.start()
        c_g.start()
        c_a.wait()
        c_r.wait()
        c_g.wait()
        xrd.wait()

        yv = a_ref[...] + b_ref[...] + r_ref[...]
        rms = jnp.sqrt(jnp.mean(yv * yv, axis=-1, keepdims=True) + 1e-6)
        o_ref[...] = yv / rms * g_ref[...]
        c_o = pltpu.make_async_copy(o_ref, out_ref.at[myrows], cp_sems.at[3])
        c_o.start()
        c_o.wait()

        def ring_rows(c):
            return pl.ds(c * CM + XEX * x, RNG)

        def my_excl_rows(c):
            return pl.ds(c * CM + RNG * x, XEX)

        def partner_excl_rows(c):
            return pl.ds(c * CM + RNG * (1 - x), XEX)

        sends = []

        def push_to_partner(c, sem_arr, h):
            p = pltpu.make_async_remote_copy(
                src_ref=out_ref.at[my_excl_rows(c)],
                dst_ref=out_ref.at[my_excl_rows(c)],
                send_sem=sem_arr.at[h],
                recv_sem=(xp_cw_recv if sem_arr is xp_cw_send
                          else xp_ccw_recv).at[h],
                device_id=partner,
                device_id_type=pl.DeviceIdType.MESH,
            )
            p.start()
            sends.append(p)

        for h in range(N_CW):
            c_cw = jnp.mod(R - h, NRING)
            s = pltpu.make_async_remote_copy(
                src_ref=out_ref.at[ring_rows(c_cw)],
                dst_ref=out_ref.at[ring_rows(c_cw)],
                send_sem=cw_send.at[h],
                recv_sem=cw_recv.at[h],
                device_id=right,
                device_id_type=pl.DeviceIdType.MESH,
            )
            s.start()
            sends.append(s)
            if h < N_CCW:
                c_ccw = jnp.mod(R + h, NRING)
                s2 = pltpu.make_async_remote_copy(
                    src_ref=out_ref.at[ring_rows(c_ccw)],
                    dst_ref=out_ref.at[ring_rows(c_ccw)],
                    send_sem=ccw_send.at[h],
                    recv_sem=ccw_recv.at[h],
                    device_id=left,
                    device_id_type=pl.DeviceIdType.MESH,
                )
                s2.start()
                sends.append(s2)

            rc = jnp.mod(R - 1 - h, NRING)
            w = pltpu.make_async_remote_copy(
                src_ref=out_ref.at[ring_rows(rc)],
                dst_ref=out_ref.at[ring_rows(rc)],
                send_sem=cw_send.at[h],
                recv_sem=cw_recv.at[h],
                device_id=left,
                device_id_type=pl.DeviceIdType.MESH,
            )
            w.wait_recv()
            push_to_partner(rc, xp_cw_send, h)
            if h < N_CCW:
                rc2 = jnp.mod(R + 1 + h, NRING)
                w2 = pltpu.make_async_remote_copy(
                    src_ref=out_ref.at[ring_rows(rc2)],
                    dst_ref=out_ref.at[ring_rows(rc2)],
                    send_sem=ccw_send.at[h],
                    recv_sem=ccw_recv.at[h],
                    device_id=right,
                    device_id_type=pl.DeviceIdType.MESH,
                )
                w2.wait_recv()
                push_to_partner(rc2, xp_ccw_send, h)

        for h in range(N_CW):
            rc = jnp.mod(R - 1 - h, NRING)
            wp = pltpu.make_async_remote_copy(
                src_ref=out_ref.at[partner_excl_rows(rc)],
                dst_ref=out_ref.at[partner_excl_rows(rc)],
                send_sem=xp_cw_send.at[h],
                recv_sem=xp_cw_recv.at[h],
                device_id=partner,
                device_id_type=pl.DeviceIdType.MESH,
            )
            wp.wait_recv()
            if h < N_CCW:
                rc2 = jnp.mod(R + 1 + h, NRING)
                wp2 = pltpu.make_async_remote_copy(
                    src_ref=out_ref.at[partner_excl_rows(rc2)],
                    dst_ref=out_ref.at[partner_excl_rows(rc2)],
                    send_sem=xp_ccw_send.at[h],
                    recv_sem=xp_ccw_recv.at[h],
                    device_id=partner,
                    device_id_type=pl.DeviceIdType.MESH,
                )
                wp2.wait_recv()

        for s in sends:
            s.wait_send()

    return pl.pallas_call(
        body,
        out_shape=jax.ShapeDtypeStruct((M, D), jnp.float32),
        in_specs=[
            pl.BlockSpec(memory_space=pl.ANY),
            pl.BlockSpec(memory_space=pl.ANY),
            pl.BlockSpec(memory_space=pl.ANY),
        ],
        out_specs=pl.BlockSpec(memory_space=pl.ANY),
        scratch_shapes=[
            pltpu.VMEM((CM, D), jnp.float32),
            pltpu.VMEM((CM, D), jnp.float32),
            pltpu.VMEM((CM, D), jnp.float32),
            pltpu.VMEM((CM, D), jnp.float32),
            pltpu.VMEM((1, D), jnp.float32),
            pltpu.SemaphoreType.DMA,
            pltpu.SemaphoreType.DMA,
            pltpu.SemaphoreType.DMA((4,)),
            pltpu.SemaphoreType.DMA((N_CW,)),
            pltpu.SemaphoreType.DMA((N_CW,)),
            pltpu.SemaphoreType.DMA((N_CCW,)),
            pltpu.SemaphoreType.DMA((N_CCW,)),
            pltpu.SemaphoreType.DMA((N_CW,)),
            pltpu.SemaphoreType.DMA((N_CW,)),
            pltpu.SemaphoreType.DMA((N_CCW,)),
            pltpu.SemaphoreType.DMA((N_CCW,)),
        ],
        compiler_params=pltpu.CompilerParams(collective_id=0),
    )(partial, resid, gamma2d)


# device time: 357613 ns/iter; 2.4977x vs baseline; 1.0822x over previous
import jax
import jax.numpy as jnp
from jax import lax
from jax.experimental import pallas as pl
from jax.experimental.pallas import tpu as pltpu

M = 8192
D = 2048
NRING = 16
CM = M // NRING
N_CW = NRING // 2
N_CCW = NRING // 2 - 1

XEX = 152
SH = CM - 2 * XEX
RNG = XEX + SH

CYC = [(0, 0), (0, 1), (0, 2), (0, 3),
       (1, 3), (1, 2), (1, 1), (2, 1),
       (2, 2), (2, 3), (3, 3), (3, 2),
       (3, 1), (3, 0), (2, 0), (1, 0)]


def kernel(partial, resid, gamma):
    gamma2d = gamma.reshape(1, D)

    def body(partial_ref, resid_ref, gamma_ref, out_ref,
             a1_ref, a2_ref, b1_ref, b2_ref, r1_ref, r2_ref,
             o1_ref, o2_ref, g_ref,
             x1_send, x1_recv, x2_send, x2_recv, cp_sems,
             cwa_send, cwa_recv, cwb_send, cwb_recv,
             ccwa_send, ccwa_recv, ccwb_send, ccwb_recv,
             xp_cw_send, xp_cw_recv, xp_ccw_send, xp_ccw_recv):
        x = lax.axis_index("x")
        yy = lax.axis_index("y")
        zz = lax.axis_index("z")

        R = jnp.int32(0)
        ry = jnp.int32(0)
        rz = jnp.int32(0)
        ly = jnp.int32(0)
        lz = jnp.int32(0)
        for k, (cy, cz) in enumerate(CYC):
            m = ((yy == cy) & (zz == cz)).astype(jnp.int32)
            nxt_y, nxt_z = CYC[(k + 1) % NRING]
            prv_y, prv_z = CYC[(k - 1) % NRING]
            R = R + k * m
            ry = ry + nxt_y * m
            rz = rz + nxt_z * m
            ly = ly + prv_y * m
            lz = lz + prv_z * m

        partner = (1 - x, yy, zz)
        right = (x, ry, rz)
        left = (x, ly, lz)

        def ring_slab(c):
            return pl.ds(c * CM + XEX * x, RNG)

        def my_excl(c):
            return pl.ds(c * CM + RNG * x, XEX)

        def partner_excl(c):
            return pl.ds(c * CM + RNG * (1 - x), XEX)

        def shared(c):
            return pl.ds(c * CM + XEX, SH)

        bar = pltpu.get_barrier_semaphore()
        for nbr in (partner, right, left):
            pl.semaphore_signal(bar, inc=1, device_id=nbr,
                                device_id_type=pl.DeviceIdType.MESH)
        pl.semaphore_wait(bar, 3)

        sends = []

        xrd1 = pltpu.make_async_remote_copy(
            src_ref=partial_ref.at[0, pl.ds(R * CM + XEX * (1 - x), RNG)],
            dst_ref=b1_ref,
            send_sem=x1_send, recv_sem=x1_recv,
            device_id=partner, device_id_type=pl.DeviceIdType.MESH)
        xrd2 = pltpu.make_async_remote_copy(
            src_ref=partial_ref.at[0, my_excl(R)], dst_ref=b2_ref,
            send_sem=x2_send, recv_sem=x2_recv,
            device_id=partner, device_id_type=pl.DeviceIdType.MESH)
        xrd1.start()
        xrd2.start()
        sends += [xrd1, xrd2]

        c_a1 = pltpu.make_async_copy(partial_ref.at[0, ring_slab(R)],
                                     a1_ref, cp_sems.at[0])
        c_r1 = pltpu.make_async_copy(resid_ref.at[ring_slab(R)],
                                     r1_ref, cp_sems.at[1])
        c_g = pltpu.make_async_copy(gamma_ref, g_ref, cp_sems.at[2])
        c_a2 = pltpu.make_async_copy(partial_ref.at[0, partner_excl(R)],
                                     a2_ref, cp_sems.at[3])
        c_r2 = pltpu.make_async_copy(resid_ref.at[partner_excl(R)],
                                     r2_ref, cp_sems.at[4])
        for c in (c_a1, c_r1, c_g, c_a2, c_r2):
            c.start()
        c_a1.wait()
        c_r1.wait()
        c_g.wait()
        xrd1.wait_recv()

        yv = a1_ref[...] + b1_ref[...] + r1_ref[...]
        rms = jnp.sqrt(jnp.mean(yv * yv, axis=-1, keepdims=True) + 1e-6)
        o1_ref[...] = yv / rms * g_ref[...]
        c_o1 = pltpu.make_async_copy(o1_ref, out_ref.at[ring_slab(R)],
                                     cp_sems.at[5])
        c_o1.start()
        c_o1.wait()

        def start_hop(h):
            c_cw = jnp.mod(R - h, NRING)
            for src, ssem, rsem in (
                    (my_excl(c_cw), cwa_send, cwa_recv),
                    (shared(c_cw), cwb_send, cwb_recv)):
                s = pltpu.make_async_remote_copy(
                    src_ref=out_ref.at[src], dst_ref=out_ref.at[src],
                    send_sem=ssem.at[h], recv_sem=rsem.at[h],
                    device_id=right, device_id_type=pl.DeviceIdType.MESH)
                s.start()
                sends.append(s)
            if h < N_CCW:
                c_ccw = jnp.mod(R + h, NRING)
                for src, ssem, rsem in (
                        (my_excl(c_ccw), ccwa_send, ccwa_recv),
                        (shared(c_ccw), ccwb_send, ccwb_recv)):
                    s = pltpu.make_async_remote_copy(
                        src_ref=out_ref.at[src], dst_ref=out_ref.at[src],
                        send_sem=ssem.at[h], recv_sem=rsem.at[h],
                        device_id=left, device_id_type=pl.DeviceIdType.MESH)
                    s.start()
                    sends.append(s)

        def wait_in(rows, ssem, rsem, h, frm):
            w = pltpu.make_async_remote_copy(
                src_ref=out_ref.at[rows], dst_ref=out_ref.at[rows],
                send_sem=ssem.at[h], recv_sem=rsem.at[h],
                device_id=frm, device_id_type=pl.DeviceIdType.MESH)
            w.wait_recv()

        def push_to_partner(c, ssem, rsem, h):
            p = pltpu.make_async_remote_copy(
                src_ref=out_ref.at[my_excl(c)],
                dst_ref=out_ref.at[my_excl(c)],
                send_sem=ssem.at[h], recv_sem=rsem.at[h],
                device_id=partner, device_id_type=pl.DeviceIdType.MESH)
            p.start()
            sends.append(p)

        start_hop(0)

        xrd2.wait_recv()
        c_a2.wait()
        c_r2.wait()
        yv2 = a2_ref[...] + b2_ref[...] + r2_ref[...]
        rms2 = jnp.sqrt(jnp.mean(yv2 * yv2, axis=-1, keepdims=True) + 1e-6)
        o2_ref[...] = yv2 / rms2 * g_ref[...]
        c_o2 = pltpu.make_async_copy(o2_ref, out_ref.at[partner_excl(R)],
                                     cp_sems.at[6])
        c_o2.start()

        for h in range(N_CW):
            if h > 0:
                start_hop(h)
            rc = jnp.mod(R - 1 - h, NRING)
            wait_in(my_excl(rc), cwa_send, cwa_recv, h, left)
            push_to_partner(rc, xp_cw_send, xp_cw_recv, h)
            wait_in(shared(rc), cwb_send, cwb_recv, h, left)
            if h < N_CCW:
                rc2 = jnp.mod(R + 1 + h, NRING)
                wait_in(my_excl(rc2), ccwa_send, ccwa_recv, h, right)
                push_to_partner(rc2, xp_ccw_send, xp_ccw_recv, h)
                wait_in(shared(rc2), ccwb_send, ccwb_recv, h, right)

        for h in range(N_CW):
            rc = jnp.mod(R - 1 - h, NRING)
            wait_in(partner_excl(rc), xp_cw_send, xp_cw_recv, h, partner)
            if h < N_CCW:
                rc2 = jnp.mod(R + 1 + h, NRING)
                wait_in(partner_excl(rc2), xp_ccw_send, xp_ccw_recv, h,
                        partner)

        c_o2.wait()
        for s in sends:
            s.wait_send()

    return pl.pallas_call(
        body,
        out_shape=jax.ShapeDtypeStruct((M, D), jnp.float32),
        in_specs=[
            pl.BlockSpec(memory_space=pl.ANY),
            pl.BlockSpec(memory_space=pl.ANY),
            pl.BlockSpec(memory_space=pl.ANY),
        ],
        out_specs=pl.BlockSpec(memory_space=pl.ANY),
        scratch_shapes=[
            pltpu.VMEM((RNG, D), jnp.float32),
            pltpu.VMEM((XEX, D), jnp.float32),
            pltpu.VMEM((RNG, D), jnp.float32),
            pltpu.VMEM((XEX, D), jnp.float32),
            pltpu.VMEM((RNG, D), jnp.float32),
            pltpu.VMEM((XEX, D), jnp.float32),
            pltpu.VMEM((RNG, D), jnp.float32),
            pltpu.VMEM((XEX, D), jnp.float32),
            pltpu.VMEM((1, D), jnp.float32),
            pltpu.SemaphoreType.DMA,
            pltpu.SemaphoreType.DMA,
            pltpu.SemaphoreType.DMA,
            pltpu.SemaphoreType.DMA,
            pltpu.SemaphoreType.DMA((7,)),
            pltpu.SemaphoreType.DMA((N_CW,)),
            pltpu.SemaphoreType.DMA((N_CW,)),
            pltpu.SemaphoreType.DMA((N_CW,)),
            pltpu.SemaphoreType.DMA((N_CW,)),
            pltpu.SemaphoreType.DMA((N_CCW,)),
            pltpu.SemaphoreType.DMA((N_CCW,)),
            pltpu.SemaphoreType.DMA((N_CCW,)),
            pltpu.SemaphoreType.DMA((N_CCW,)),
            pltpu.SemaphoreType.DMA((N_CW,)),
            pltpu.SemaphoreType.DMA((N_CW,)),
            pltpu.SemaphoreType.DMA((N_CCW,)),
            pltpu.SemaphoreType.DMA((N_CCW,)),
        ],
        compiler_params=pltpu.CompilerParams(collective_id=0),
    )(partial, resid, gamma2d)


# device time: 342780 ns/iter; 2.6058x vs baseline; 1.0433x over previous
import jax
import jax.numpy as jnp
from jax import lax
from jax.experimental import pallas as pl
from jax.experimental.pallas import tpu as pltpu

M = 8192
D = 2048
NRING = 16
CM = M // NRING
N_CW = NRING // 2
N_CCW = NRING // 2 - 1

XEX = 152
SH = CM - 2 * XEX
RNG = XEX + SH

CYC = [(0, 0), (0, 1), (0, 2), (0, 3),
       (1, 3), (1, 2), (1, 1), (2, 1),
       (2, 2), (2, 3), (3, 3), (3, 2),
       (3, 1), (3, 0), (2, 0), (1, 0)]


def kernel(partial, resid, gamma):
    gamma2d = gamma.reshape(1, D)

    def body(partial_ref, resid_ref, gamma_ref, out_ref,
             aa_ref, ab_ref, ar_ref, ba_ref, bb_ref, br_ref,
             ra_ref, rb_ref, rr_ref, oa_ref, ob_ref, or_ref, g_ref,
             xa_send, xa_recv, xb_send, xb_recv, xr_send, xr_recv,
             cp_sems,
             cwa_send, cwa_recv, cwb_send, cwb_recv,
             ccwa_send, ccwa_recv, ccwb_send, ccwb_recv,
             xp_cw_send, xp_cw_recv, xp_ccw_send, xp_ccw_recv):
        x = lax.axis_index("x")
        yy = lax.axis_index("y")
        zz = lax.axis_index("z")

        R = jnp.int32(0)
        ry = jnp.int32(0)
        rz = jnp.int32(0)
        ly = jnp.int32(0)
        lz = jnp.int32(0)
        for k, (cy, cz) in enumerate(CYC):
            m = ((yy == cy) & (zz == cz)).astype(jnp.int32)
            nxt_y, nxt_z = CYC[(k + 1) % NRING]
            prv_y, prv_z = CYC[(k - 1) % NRING]
            R = R + k * m
            ry = ry + nxt_y * m
            rz = rz + nxt_z * m
            ly = ly + prv_y * m
            lz = lz + prv_z * m

        partner = (1 - x, yy, zz)
        right = (x, ry, rz)
        left = (x, ly, lz)

        def my_excl(c):
            return pl.ds(c * CM + RNG * x, XEX)

        def partner_excl(c):
            return pl.ds(c * CM + RNG * (1 - x), XEX)

        def shared(c):
            return pl.ds(c * CM + XEX, SH)

        bar = pltpu.get_barrier_semaphore()
        for nbr in (partner, right, left):
            pl.semaphore_signal(bar, inc=1, device_id=nbr,
                                device_id_type=pl.DeviceIdType.MESH)
        pl.semaphore_wait(bar, 3)

        sends = []

        xrd_a = pltpu.make_async_remote_copy(
            src_ref=partial_ref.at[0, partner_excl(R)], dst_ref=ba_ref,
            send_sem=xa_send, recv_sem=xa_recv,
            device_id=partner, device_id_type=pl.DeviceIdType.MESH)
        xrd_b = pltpu.make_async_remote_copy(
            src_ref=partial_ref.at[0, shared(R)], dst_ref=bb_ref,
            send_sem=xb_send, recv_sem=xb_recv,
            device_id=partner, device_id_type=pl.DeviceIdType.MESH)
        xrd_r = pltpu.make_async_remote_copy(
            src_ref=partial_ref.at[0, my_excl(R)], dst_ref=br_ref,
            send_sem=xr_send, recv_sem=xr_recv,
            device_id=partner, device_id_type=pl.DeviceIdType.MESH)
        xrd_a.start()
        xrd_b.start()
        xrd_r.start()
        sends += [xrd_a, xrd_b, xrd_r]

        c_aa = pltpu.make_async_copy(partial_ref.at[0, my_excl(R)],
                                     aa_ref, cp_sems.at[0])
        c_ra = pltpu.make_async_copy(resid_ref.at[my_excl(R)],
                                     ra_ref, cp_sems.at[1])
        c_g = pltpu.make_async_copy(gamma_ref, g_ref, cp_sems.at[2])
        c_ab = pltpu.make_async_copy(partial_ref.at[0, shared(R)],
                                     ab_ref, cp_sems.at[3])
        c_rb = pltpu.make_async_copy(resid_ref.at[shared(R)],
                                     rb_ref, cp_sems.at[4])
        c_ar = pltpu.make_async_copy(partial_ref.at[0, partner_excl(R)],
                                     ar_ref, cp_sems.at[5])
        c_rr = pltpu.make_async_copy(resid_ref.at[partner_excl(R)],
                                     rr_ref, cp_sems.at[6])
        for c in (c_aa, c_ra, c_g, c_ab, c_rb, c_ar, c_rr):
            c.start()

        def rmsnorm(a, b, r):
            yv = a[...] + b[...] + r[...]
            rms = jnp.sqrt(jnp.mean(yv * yv, axis=-1, keepdims=True) + 1e-6)
            return yv / rms * g_ref[...]

        def start_sub(c_cw, c_ccw, h, sub):
            ssem, rsem = {"a": (cwa_send, cwa_recv),
                          "b": (cwb_send, cwb_recv)}[sub]
            rows = my_excl(c_cw) if sub == "a" else shared(c_cw)
            s = pltpu.make_async_remote_copy(
                src_ref=out_ref.at[rows], dst_ref=out_ref.at[rows],
                send_sem=ssem.at[h], recv_sem=rsem.at[h],
                device_id=right, device_id_type=pl.DeviceIdType.MESH)
            s.start()
            sends.append(s)
            if h < N_CCW:
                ssem, rsem = {"a": (ccwa_send, ccwa_recv),
                              "b": (ccwb_send, ccwb_recv)}[sub]
                rows = my_excl(c_ccw) if sub == "a" else shared(c_ccw)
                s = pltpu.make_async_remote_copy(
                    src_ref=out_ref.at[rows], dst_ref=out_ref.at[rows],
                    send_sem=ssem.at[h], recv_sem=rsem.at[h],
                    device_id=left, device_id_type=pl.DeviceIdType.MESH)
                s.start()
                sends.append(s)

        def wait_in(rows, ssem, rsem, h, frm):
            w = pltpu.make_async_remote_copy(
                src_ref=out_ref.at[rows], dst_ref=out_ref.at[rows],
                send_sem=ssem.at[h], recv_sem=rsem.at[h],
                device_id=frm, device_id_type=pl.DeviceIdType.MESH)
            w.wait_recv()

        def push_to_partner(c, ssem, rsem, h):
            p = pltpu.make_async_remote_copy(
                src_ref=out_ref.at[my_excl(c)],
                dst_ref=out_ref.at[my_excl(c)],
                send_sem=ssem.at[h], recv_sem=rsem.at[h],
                device_id=partner, device_id_type=pl.DeviceIdType.MESH)
            p.start()
            sends.append(p)

        c_aa.wait()
        c_ra.wait()
        c_g.wait()
        xrd_a.wait_recv()
        oa_ref[...] = rmsnorm(aa_ref, ba_ref, ra_ref)
        c_oa = pltpu.make_async_copy(oa_ref, out_ref.at[my_excl(R)],
                                     cp_sems.at[7])
        c_oa.start()
        c_oa.wait()
        start_sub(R, R, 0, "a")

        c_ab.wait()
        c_rb.wait()
        xrd_b.wait_recv()
        ob_ref[...] = rmsnorm(ab_ref, bb_ref, rb_ref)
        c_ob = pltpu.make_async_copy(ob_ref, out_ref.at[shared(R)],
                                     cp_sems.at[8])
        c_ob.start()
        c_ob.wait()
        start_sub(R, R, 0, "b")

        c_ar.wait()
        c_rr.wait()
        xrd_r.wait_recv()
        or_ref[...] = rmsnorm(ar_ref, br_ref, rr_ref)
        c_or = pltpu.make_async_copy(or_ref, out_ref.at[partner_excl(R)],
                                     cp_sems.at[9])
        c_or.start()

        for h in range(N_CW):
            if h > 0:
                c_cw = jnp.mod(R - h, NRING)
                c_ccw = jnp.mod(R + h, NRING)
                start_sub(c_cw, c_ccw, h, "a")
                start_sub(c_cw, c_ccw, h, "b")
            rc = jnp.mod(R - 1 - h, NRING)
            rc2 = jnp.mod(R + 1 + h, NRING)
            wait_in(my_excl(rc), cwa_send, cwa_recv, h, left)
            push_to_partner(rc, xp_cw_send, xp_cw_recv, h)
            if h < N_CCW:
                wait_in(my_excl(rc2), ccwa_send, ccwa_recv, h, right)
                push_to_partner(rc2, xp_ccw_send, xp_ccw_recv, h)
            wait_in(shared(rc), cwb_send, cwb_recv, h, left)
            if h < N_CCW:
                wait_in(shared(rc2), ccwb_send, ccwb_recv, h, right)

        for h in range(N_CW):
            rc = jnp.mod(R - 1 - h, NRING)
            wait_in(partner_excl(rc), xp_cw_send, xp_cw_recv, h, partner)
            if h < N_CCW:
                rc2 = jnp.mod(R + 1 + h, NRING)
                wait_in(partner_excl(rc2), xp_ccw_send, xp_ccw_recv, h,
                        partner)

        c_or.wait()
        for s in sends:
            s.wait_send()

    return pl.pallas_call(
        body,
        out_shape=jax.ShapeDtypeStruct((M, D), jnp.float32),
        in_specs=[
            pl.BlockSpec(memory_space=pl.ANY),
            pl.BlockSpec(memory_space=pl.ANY),
            pl.BlockSpec(memory_space=pl.ANY),
        ],
        out_specs=pl.BlockSpec(memory_space=pl.ANY),
        scratch_shapes=[
            pltpu.VMEM((XEX, D), jnp.float32),
            pltpu.VMEM((SH, D), jnp.float32),
            pltpu.VMEM((XEX, D), jnp.float32),
            pltpu.VMEM((XEX, D), jnp.float32),
            pltpu.VMEM((SH, D), jnp.float32),
            pltpu.VMEM((XEX, D), jnp.float32),
            pltpu.VMEM((XEX, D), jnp.float32),
            pltpu.VMEM((SH, D), jnp.float32),
            pltpu.VMEM((XEX, D), jnp.float32),
            pltpu.VMEM((XEX, D), jnp.float32),
            pltpu.VMEM((SH, D), jnp.float32),
            pltpu.VMEM((XEX, D), jnp.float32),
            pltpu.VMEM((1, D), jnp.float32),
            pltpu.SemaphoreType.DMA,
            pltpu.SemaphoreType.DMA,
            pltpu.SemaphoreType.DMA,
            pltpu.SemaphoreType.DMA,
            pltpu.SemaphoreType.DMA,
            pltpu.SemaphoreType.DMA,
            pltpu.SemaphoreType.DMA((10,)),
            pltpu.SemaphoreType.DMA((N_CW,)),
            pltpu.SemaphoreType.DMA((N_CW,)),
            pltpu.SemaphoreType.DMA((N_CW,)),
            pltpu.SemaphoreType.DMA((N_CW,)),
            pltpu.SemaphoreType.DMA((N_CCW,)),
            pltpu.SemaphoreType.DMA((N_CCW,)),
            pltpu.SemaphoreType.DMA((N_CCW,)),
            pltpu.SemaphoreType.DMA((N_CCW,)),
            pltpu.SemaphoreType.DMA((N_CW,)),
            pltpu.SemaphoreType.DMA((N_CW,)),
            pltpu.SemaphoreType.DMA((N_CCW,)),
            pltpu.SemaphoreType.DMA((N_CCW,)),
        ],
        compiler_params=pltpu.CompilerParams(collective_id=0),
    )(partial, resid, gamma2d)


# device time: 326277 ns/iter; 2.7375x vs baseline; 1.0506x over previous
import jax
import jax.numpy as jnp
from jax import lax
from jax.experimental import pallas as pl
from jax.experimental.pallas import tpu as pltpu

M = 8192
D = 2048
NRING = 16
CM = M // NRING
N_CW = NRING // 2
N_CCW = NRING // 2 - 1

XEX = 152
SH = CM - 2 * XEX
RNG = XEX + SH

CYC = [(0, 0), (0, 1), (0, 2), (0, 3),
       (1, 3), (1, 2), (1, 1), (2, 1),
       (2, 2), (2, 3), (3, 3), (3, 2),
       (3, 1), (3, 0), (2, 0), (1, 0)]


def kernel(partial, resid, gamma):
    gamma2d = gamma.reshape(1, D)

    def body(partial_ref, resid_ref, gamma_ref, out_ref,
             aa_ref, ab_ref, ar_ref, ba_ref, bb_ref, br_ref,
             ra_ref, rb_ref, rr_ref, oa_ref, ob_ref, or_ref, g_ref,
             xa_send, xa_recv, xb_send, xb_recv, xr_send, xr_recv,
             cp_sems,
             cwa_send, cwa_recv, cwb_send, cwb_recv,
             ccwa_send, ccwa_recv, ccwb_send, ccwb_recv,
             xp_cw_send, xp_cw_recv, xp_ccw_send, xp_ccw_recv):
        x = lax.axis_index("x")
        yy = lax.axis_index("y")
        zz = lax.axis_index("z")

        R = jnp.int32(0)
        ry = jnp.int32(0)
        rz = jnp.int32(0)
        ly = jnp.int32(0)
        lz = jnp.int32(0)
        for k, (cy, cz) in enumerate(CYC):
            m = ((yy == cy) & (zz == cz)).astype(jnp.int32)
            nxt_y, nxt_z = CYC[(k + 1) % NRING]
            prv_y, prv_z = CYC[(k - 1) % NRING]
            R = R + k * m
            ry = ry + nxt_y * m
            rz = rz + nxt_z * m
            ly = ly + prv_y * m
            lz = lz + prv_z * m

        partner = (1 - x, yy, zz)
        right = (x, ry, rz)
        left = (x, ly, lz)

        def my_excl(c):
            return pl.ds(c * CM + RNG * x, XEX)

        def partner_excl(c):
            return pl.ds(c * CM + RNG * (1 - x), XEX)

        def shared(c):
            return pl.ds(c * CM + XEX, SH)

        bar = pltpu.get_barrier_semaphore()
        for nbr in (partner, right, left):
            pl.semaphore_signal(bar, inc=1, device_id=nbr,
                                device_id_type=pl.DeviceIdType.MESH)
        pl.semaphore_wait(bar, 3)

        sends = []

        xrd_a = pltpu.make_async_remote_copy(
            src_ref=partial_ref.at[0, partner_excl(R)], dst_ref=ba_ref,
            send_sem=xa_send, recv_sem=xa_recv,
            device_id=partner, device_id_type=pl.DeviceIdType.MESH)
        xrd_b = pltpu.make_async_remote_copy(
            src_ref=partial_ref.at[0, shared(R)], dst_ref=bb_ref,
            send_sem=xb_send, recv_sem=xb_recv,
            device_id=partner, device_id_type=pl.DeviceIdType.MESH)
        xrd_r = pltpu.make_async_remote_copy(
            src_ref=partial_ref.at[0, my_excl(R)], dst_ref=br_ref,
            send_sem=xr_send, recv_sem=xr_recv,
            device_id=partner, device_id_type=pl.DeviceIdType.MESH)
        xrd_a.start()
        xrd_b.start()
        xrd_r.start()
        sends += [xrd_a, xrd_b, xrd_r]

        c_aa = pltpu.make_async_copy(partial_ref.at[0, my_excl(R)],
                                     aa_ref, cp_sems.at[0])
        c_ra = pltpu.make_async_copy(resid_ref.at[my_excl(R)],
                                     ra_ref, cp_sems.at[1])
        c_g = pltpu.make_async_copy(gamma_ref, g_ref, cp_sems.at[2])
        c_ab = pltpu.make_async_copy(partial_ref.at[0, shared(R)],
                                     ab_ref, cp_sems.at[3])
        c_rb = pltpu.make_async_copy(resid_ref.at[shared(R)],
                                     rb_ref, cp_sems.at[4])
        c_ar = pltpu.make_async_copy(partial_ref.at[0, partner_excl(R)],
                                     ar_ref, cp_sems.at[5])
        c_rr = pltpu.make_async_copy(resid_ref.at[partner_excl(R)],
                                     rr_ref, cp_sems.at[6])
        for c in (c_aa, c_ra, c_g, c_ab, c_rb, c_ar, c_rr):
            c.start()

        def rmsnorm(a, b, r):
            yv = a[...] + b[...] + r[...]
            rms = jnp.sqrt(jnp.mean(yv * yv, axis=-1, keepdims=True) + 1e-6)
            return yv / rms * g_ref[...]

        def start_one(c, h, sub, direction):
            if direction == "cw":
                ssem, rsem = {"a": (cwa_send, cwa_recv),
                              "b": (cwb_send, cwb_recv)}[sub]
                tgt = right
            else:
                ssem, rsem = {"a": (ccwa_send, ccwa_recv),
                              "b": (ccwb_send, ccwb_recv)}[sub]
                tgt = left
            rows = my_excl(c) if sub == "a" else shared(c)
            s = pltpu.make_async_remote_copy(
                src_ref=out_ref.at[rows], dst_ref=out_ref.at[rows],
                send_sem=ssem.at[h], recv_sem=rsem.at[h],
                device_id=tgt, device_id_type=pl.DeviceIdType.MESH)
            s.start()
            sends.append(s)

        def wait_in(rows, ssem, rsem, h, frm):
            w = pltpu.make_async_remote_copy(
                src_ref=out_ref.at[rows], dst_ref=out_ref.at[rows],
                send_sem=ssem.at[h], recv_sem=rsem.at[h],
                device_id=frm, device_id_type=pl.DeviceIdType.MESH)
            w.wait_recv()

        def push_to_partner(c, ssem, rsem, h):
            p = pltpu.make_async_remote_copy(
                src_ref=out_ref.at[my_excl(c)],
                dst_ref=out_ref.at[my_excl(c)],
                send_sem=ssem.at[h], recv_sem=rsem.at[h],
                device_id=partner, device_id_type=pl.DeviceIdType.MESH)
            p.start()
            sends.append(p)

        c_aa.wait()
        c_ra.wait()
        c_g.wait()
        xrd_a.wait_recv()
        oa_ref[...] = rmsnorm(aa_ref, ba_ref, ra_ref)
        c_oa = pltpu.make_async_copy(oa_ref, out_ref.at[my_excl(R)],
                                     cp_sems.at[7])
        c_oa.start()
        c_oa.wait()
        start_one(R, 0, "a", "cw")
        start_one(R, 0, "a", "ccw")

        c_ab.wait()
        c_rb.wait()
        xrd_b.wait_recv()
        ob_ref[...] = rmsnorm(ab_ref, bb_ref, rb_ref)
        c_ob = pltpu.make_async_copy(ob_ref, out_ref.at[shared(R)],
                                     cp_sems.at[8])
        c_ob.start()
        c_ob.wait()
        start_one(R, 0, "b", "cw")
        start_one(R, 0, "b", "ccw")

        c_ar.wait()
        c_rr.wait()
        xrd_r.wait_recv()
        or_ref[...] = rmsnorm(ar_ref, br_ref, rr_ref)
        c_or = pltpu.make_async_copy(or_ref, out_ref.at[partner_excl(R)],
                                     cp_sems.at[9])
        c_or.start()

        for h in range(N_CW):
            rc = jnp.mod(R - 1 - h, NRING)
            rc2 = jnp.mod(R + 1 + h, NRING)
            wait_in(my_excl(rc), cwa_send, cwa_recv, h, left)
            if h + 1 < N_CW:
                start_one(rc, h + 1, "a", "cw")
            push_to_partner(rc, xp_cw_send, xp_cw_recv, h)
            if h < N_CCW:
                wait_in(my_excl(rc2), ccwa_send, ccwa_recv, h, right)
                if h + 1 < N_CCW:
                    start_one(rc2, h + 1, "a", "ccw")
                push_to_partner(rc2, xp_ccw_send, xp_ccw_recv, h)
            wait_in(shared(rc), cwb_send, cwb_recv, h, left)
            if h + 1 < N_CW:
                start_one(rc, h + 1, "b", "cw")
            if h < N_CCW:
                wait_in(shared(rc2), ccwb_send, ccwb_recv, h, right)
                if h + 1 < N_CCW:
                    start_one(rc2, h + 1, "b", "ccw")

        for h in range(N_CW):
            rc = jnp.mod(R - 1 - h, NRING)
            wait_in(partner_excl(rc), xp_cw_send, xp_cw_recv, h, partner)
            if h < N_CCW:
                rc2 = jnp.mod(R + 1 + h, NRING)
                wait_in(partner_excl(rc2), xp_ccw_send, xp_ccw_recv, h,
                        partner)

        c_or.wait()
        for s in sends:
            s.wait_send()

    return pl.pallas_call(
        body,
        out_shape=jax.ShapeDtypeStruct((M, D), jnp.float32),
        in_specs=[
            pl.BlockSpec(memory_space=pl.ANY),
            pl.BlockSpec(memory_space=pl.ANY),
            pl.BlockSpec(memory_space=pl.ANY),
        ],
        out_specs=pl.BlockSpec(memory_space=pl.ANY),
        scratch_shapes=[
            pltpu.VMEM((XEX, D), jnp.float32),
            pltpu.VMEM((SH, D), jnp.float32),
            pltpu.VMEM((XEX, D), jnp.float32),
            pltpu.VMEM((XEX, D), jnp.float32),
            pltpu.VMEM((SH, D), jnp.float32),
            pltpu.VMEM((XEX, D), jnp.float32),
            pltpu.VMEM((XEX, D), jnp.float32),
            pltpu.VMEM((SH, D), jnp.float32),
            pltpu.VMEM((XEX, D), jnp.float32),
            pltpu.VMEM((XEX, D), jnp.float32),
            pltpu.VMEM((SH, D), jnp.float32),
            pltpu.VMEM((XEX, D), jnp.float32),
            pltpu.VMEM((1, D), jnp.float32),
            pltpu.SemaphoreType.DMA,
            pltpu.SemaphoreType.DMA,
            pltpu.SemaphoreType.DMA,
            pltpu.SemaphoreType.DMA,
            pltpu.SemaphoreType.DMA,
            pltpu.SemaphoreType.DMA,
            pltpu.SemaphoreType.DMA((10,)),
            pltpu.SemaphoreType.DMA((N_CW,)),
            pltpu.SemaphoreType.DMA((N_CW,)),
            pltpu.SemaphoreType.DMA((N_CW,)),
            pltpu.SemaphoreType.DMA((N_CW,)),
            pltpu.SemaphoreType.DMA((N_CCW,)),
            pltpu.SemaphoreType.DMA((N_CCW,)),
            pltpu.SemaphoreType.DMA((N_CCW,)),
            pltpu.SemaphoreType.DMA((N_CCW,)),
            pltpu.SemaphoreType.DMA((N_CW,)),
            pltpu.SemaphoreType.DMA((N_CW,)),
            pltpu.SemaphoreType.DMA((N_CCW,)),
            pltpu.SemaphoreType.DMA((N_CCW,)),
        ],
        compiler_params=pltpu.CompilerParams(collective_id=0),
    )(partial, resid, gamma2d)
